# Initial kernel scaffold; baseline (speedup 1.0000x reference)
import functools

import jax
import jax.numpy as jnp
from jax import lax
from jax.experimental import pallas as pl
from jax.experimental.pallas import tpu as pltpu
from jax.experimental.pallas import tpu_sc as plsc

D = 64
NC = 2
NS = 16
NW = NC * NS
C = 128
NBUF = 8


@functools.partial(jax.jit, static_argnames=("nch",))
def _embed_gather(x_r, table, *, nch):
    b_total = NW * nch * C
    mesh = plsc.VectorSubcoreMesh(core_axis_name="c", subcore_axis_name="s")

    def body(x_hbm, table_hbm, out_hbm, idx_v, *rest):
        rows = rest[:NBUF]
        gsem = rest[NBUF:2 * NBUF]
        osem = rest[2 * NBUF:3 * NBUF]
        wid = lax.axis_index("s") * NC + lax.axis_index("c")
        base = wid * (nch * C)

        pltpu.sync_copy(x_hbm.at[wid], idx_v)

        def gather_start(j, b):
            pltpu.async_copy(table_hbm.at[idx_v.at[j]], rows[b], gsem[b])

        def gather_wait(j, b):
            pltpu.make_async_copy(
                table_hbm.at[idx_v.at[j]], rows[b], gsem[b]).wait()

        def out_start(j, b):
            pltpu.async_copy(
                rows[b], out_hbm.at[pl.ds(base + j * C, C)], osem[b])

        def out_wait(j, b):
            pltpu.make_async_copy(
                rows[b], out_hbm.at[pl.ds(base + j * C, C)], osem[b]).wait()

        for b in range(NBUF):
            gather_start(b, b)

        @pl.loop(0, nch, step=NBUF)
        def _outer(i):
            for b in range(NBUF):
                j = i + b
                gather_wait(j, b)
                out_start(j, b)
            for b in range(NBUF):
                j = i + b
                out_wait(j, b)

                @pl.when(j + NBUF < nch)
                def _():
                    gather_start(j + NBUF, b)

    call = pl.kernel(
        body,
        out_type=jax.ShapeDtypeStruct((b_total, D), jnp.float32),
        mesh=mesh,
        scratch_types=(
            [pltpu.VMEM((nch, C), jnp.int32)]
            + [pltpu.VMEM((C, D), jnp.float32) for _ in range(NBUF)]
            + [pltpu.SemaphoreType.DMA for _ in range(2 * NBUF)]
        ),
    )
    return call(x_r, table)


def kernel(x, phychem):
    n, s = x.shape
    b_total = n * s
    nch = b_total // (NW * C)
    x_r = x.reshape(NW, nch, C)
    out = _embed_gather(x_r, phychem, nch=nch)
    return out.reshape(n, s, D)


# speedup vs baseline: 6.2210x; 6.2210x over previous
import functools

import jax
import jax.numpy as jnp
from jax import lax
from jax.experimental import pallas as pl
from jax.experimental.pallas import tpu as pltpu
from jax.experimental.pallas import tpu_sc as plsc

D = 64
NC = 2
NS = 16
NW = NC * NS
C = 128
NBUF = 8


@functools.partial(jax.jit, static_argnames=("nch",))
def _embed_gather(x_r, table, *, nch):
    b_total = NW * nch * C
    mesh = plsc.VectorSubcoreMesh(core_axis_name="c", subcore_axis_name="s")

    def body(x_hbm, table_hbm, out_hbm, idx_v, *rest):
        rows = rest[:NBUF]
        gsem = rest[NBUF:2 * NBUF]
        osem = rest[2 * NBUF:3 * NBUF]
        wid = lax.axis_index("s") * NC + lax.axis_index("c")
        base = wid * (nch * C)

        pltpu.sync_copy(x_hbm.at[wid], idx_v)

        def gather_start(j, b):
            pltpu.async_copy(table_hbm.at[idx_v.at[j]], rows[b], gsem[b])

        def gather_wait(j, b):
            pltpu.make_async_copy(
                table_hbm.at[idx_v.at[j]], rows[b], gsem[b]).wait()

        def out_start(j, b):
            pltpu.async_copy(
                rows[b], out_hbm.at[pl.ds(base + j * C, C)], osem[b])

        def out_wait(j, b):
            pltpu.make_async_copy(
                rows[b], out_hbm.at[pl.ds(base + j * C, C)], osem[b]).wait()

        for b in range(NBUF):
            gather_start(b, b)

        @pl.loop(0, nch, step=NBUF)
        def _outer(i):
            for b in range(NBUF):
                j = i + b
                gather_wait(j, b)
                out_start(j, b)
            for b in range(NBUF):
                j = i + b
                out_wait(j, b)

                @pl.when(j + NBUF < nch)
                def _():
                    gather_start(j + NBUF, b)

    call = pl.kernel(
        body,
        out_type=jax.ShapeDtypeStruct((b_total, D), jnp.float32),
        mesh=mesh,
        scratch_types=(
            [pltpu.VMEM((nch, C), jnp.int32)]
            + [pltpu.VMEM((C, D), jnp.float32) for _ in range(NBUF)]
            + [pltpu.SemaphoreType.DMA for _ in range(2 * NBUF)]
        ),
        compiler_params=pltpu.CompilerParams(use_tc_tiling_on_sc=False),
    )
    return call(x_r, table)


def kernel(x, phychem):
    n, s = x.shape
    b_total = n * s
    nch = b_total // (NW * C)
    x_r = x.reshape(NW, nch, C)
    out = _embed_gather(x_r, phychem, nch=nch)
    return out.reshape(n, s, D)


# speedup vs baseline: 10.9474x; 1.7597x over previous
import functools

import jax
import jax.numpy as jnp
from jax import lax
from jax.experimental import pallas as pl
from jax.experimental.pallas import tpu as pltpu
from jax.experimental.pallas import tpu_sc as plsc

D = 64
SP = 56
DP = 128
NC = 2
NS = 16
NW = NC * NS
NBUF = 8


@functools.partial(jax.jit, static_argnames=("n", "s"))
def _embed_gather(x, table, *, n, s):
    npw = n // NW
    mesh = plsc.VectorSubcoreMesh(core_axis_name="c", subcore_axis_name="s")

    def body(x_hbm, table_hbm, out_hbm, idx_v, *rest):
        rows = rest[:NBUF]
        gsem = rest[NBUF:2 * NBUF]
        osem = rest[2 * NBUF:3 * NBUF]
        wid = lax.axis_index("s") * NC + lax.axis_index("c")
        base_n = wid * npw

        pltpu.sync_copy(x_hbm.at[pl.ds(base_n, npw)], idx_v)

        def g_start(j, b):
            pltpu.async_copy(table_hbm.at[idx_v.at[j]], rows[b], gsem[b])

        def g_wait(j, b):
            pltpu.make_async_copy(
                table_hbm.at[idx_v.at[j]], rows[b], gsem[b]).wait()

        def o_start(j, b):
            pltpu.async_copy(
                rows[b],
                out_hbm.at[base_n + j, pl.ds(0, s), pl.ds(0, D)], osem[b])

        def o_wait(j, b):
            pltpu.make_async_copy(
                rows[b],
                out_hbm.at[base_n + j, pl.ds(0, s), pl.ds(0, D)],
                osem[b]).wait()

        for b in range(NBUF):
            g_start(b, b)

        @pl.loop(0, npw, step=NBUF)
        def _outer(i):
            for b in range(NBUF):
                j = i + b
                g_wait(j, b)
                o_start(j, b)
            for b in range(NBUF):
                j = i + b
                o_wait(j, b)

                @pl.when(j + NBUF < npw)
                def _():
                    g_start(j + NBUF, b)

    call = pl.kernel(
        body,
        out_type=jax.ShapeDtypeStruct((n, SP, DP), jnp.float32),
        mesh=mesh,
        scratch_types=(
            [pltpu.VMEM((npw, s), jnp.int32)]
            + [pltpu.VMEM((s, D), jnp.float32) for _ in range(NBUF)]
            + [pltpu.SemaphoreType.DMA for _ in range(2 * NBUF)]
        ),
        compiler_params=pltpu.CompilerParams(use_tc_tiling_on_sc=False),
    )
    return call(x, table)


def kernel(x, phychem):
    n, s = x.shape
    out_p = _embed_gather(x, phychem, n=n, s=s)
    return out_p[:, :s, :D]
